# baseline (speedup 1.0000x reference)
import functools

import jax
import jax.numpy as jnp
import numpy as np
from jax import lax
from jax.experimental import pallas as pl
from jax.experimental.pallas import tpu as pltpu
from jax.experimental.pallas import tpu_sc as plsc

N = 10000
E = 160000
K = 3
NG = 64
NC = 2
H = 128

NCORE = 2
NSUB = 16
NW = NCORE * NSUB
CHR = E // NW
CH = 5008
NP = 10240
PCH = 5120
LEAK = 0.01

_mesh = plsc.VectorSubcoreMesh(core_axis_name="c", subcore_axis_name="s")
_sc_params = pltpu.CompilerParams(needs_layout_passes=False)


def _leaky(x):
    return jnp.where(x >= 0, x, LEAK * x)


def _dgt(a, b):
    return lax.dot_general(a, b, (((0,), (0,)), ((), ())),
                           preferred_element_type=jnp.float32)


def _node_mlp_body(x_ref, w1_ref, b1_ref, w2_ref, b2_ref, o_ref):
    x = x_ref[...]
    h = jnp.dot(x, w1_ref[...], preferred_element_type=jnp.float32) + b1_ref[...]
    m = jnp.mean(h, axis=0, keepdims=True)
    v = jnp.mean((h - m) ** 2, axis=0, keepdims=True)
    hn = _leaky((h - m) * lax.rsqrt(v + 1e-5))
    o = lax.dot_general(w2_ref[...], hn, (((0,), (1,)), ((), ())),
                        preferred_element_type=jnp.float32) + b2_ref[...]
    o_ref[...] = jnp.tanh(o) * np.pi


_EB8 = 2000
_NB8 = (E // 8) // _EB8


def _edge_mlp_body(x_ref, w1_ref, b1_ref, w2_ref, b2_ref, o_ref, stat_ref):
    i = pl.program_id(0)
    j = pl.program_id(1)

    @pl.when(jnp.logical_and(i == 0, j == 0))
    def _():
        stat_ref[...] = jnp.zeros_like(stat_ref)

    h = (jnp.dot(x_ref[...], w1_ref[...], preferred_element_type=jnp.float32)
         + b1_ref[...])

    @pl.when(i == 0)
    def _():
        stat_ref[0:1] += jnp.sum(h, axis=0, keepdims=True)
        stat_ref[1:2] += jnp.sum(h * h, axis=0, keepdims=True)
        o_ref[...] = jnp.zeros_like(o_ref)

    @pl.when(jnp.logical_and(i == 0, j == _NB8 - 1))
    def _():
        sm = jnp.sum(stat_ref[0:1].reshape(8, H), axis=0, keepdims=True) / E
        ss = jnp.sum(stat_ref[1:2].reshape(8, H), axis=0, keepdims=True) / E
        var = ss - sm * sm
        rs = lax.rsqrt(var + 1e-5)
        stat_ref[2:3] = jnp.concatenate([sm] * 8, axis=1)
        stat_ref[3:4] = jnp.concatenate([rs] * 8, axis=1)

    @pl.when(i == 1)
    def _():
        hn = _leaky((h - stat_ref[2:3]) * stat_ref[3:4])
        o = (jnp.dot(hn, w2_ref[...], preferred_element_type=jnp.float32)
             + b2_ref[...])
        o_ref[...] = jnp.tanh(o) * np.pi


@functools.partial(
    pl.kernel,
    out_type=(
        jax.ShapeDtypeStruct((NW, 1, NP), jnp.int32),
        jax.ShapeDtypeStruct((NW, 1, CH), jnp.int32),
    ),
    mesh=_mesh,
    scratch_types=[
        pltpu.VMEM((CH,), jnp.int32),
        pltpu.VMEM((NP,), jnp.int32),
        pltpu.VMEM((CH,), jnp.int32),
    ],
    compiler_params=_sc_params,
)
def _sc_hist(dst_hbm, cnt_hbm, lrank_hbm, dstv, cntv, lrankv):
    cid = lax.axis_index("c")
    sid = lax.axis_index("s")
    wid = cid * NSUB + sid
    pltpu.sync_copy(dst_hbm.at[wid, 0], dstv)

    def zero_body(i, c):
        cntv[pl.ds(i * 16, 16)] = jnp.zeros((16,), jnp.int32)
        return c

    lax.fori_loop(0, NP // 16, zero_body, 0)

    def body(i, c):
        base = i * 16
        d = dstv[pl.ds(base, 16)]
        r0 = plsc.load_gather(cntv, [d])
        cntd, _ = plsc.scan_count(d)
        lrankv[pl.ds(base, 16)] = d * 8192 + (r0 + cntd - 1)
        plsc.store_scatter(cntv, [d], r0 + cntd)
        return c

    lax.fori_loop(0, CH // 16, body, 0)
    pltpu.sync_copy(cntv, cnt_hbm.at[wid, 0])
    pltpu.sync_copy(lrankv, lrank_hbm.at[wid, 0])


SL = NP // NSUB


@functools.partial(
    pl.kernel,
    out_type=jax.ShapeDtypeStruct((NW, 1, 3 * NP), jnp.float32),
    mesh=_mesh,
    scratch_types=[
        pltpu.VMEM((NP,), jnp.int32),
        pltpu.VMEM((CH,), jnp.int32),
        pltpu.VMEM((CH,), jnp.int32),
        pltpu.VMEM((CH,), jnp.int32),
        pltpu.VMEM((N,), jnp.int32),
        pltpu.VMEM((48,), jnp.float32),
        pltpu.VMEM((3 * NP,), jnp.float32),
        pltpu.VMEM((NSUB * SL,), jnp.int32),
        pltpu.VMEM((NSUB * SL,), jnp.int32),
        pltpu.VMEM((NSUB * SL,), jnp.int32),
        pltpu.VMEM_SHARED((NSUB, 1, NSUB * SL), jnp.int32),
    ],
    compiler_params=_sc_params,
)
def _sc_accum(cnt_hbm, lrank_hbm, src_hbm, e01_hbm, n01_hbm,
              tab_hbm, acc_hbm, basev, srcv, cmbv,
              e01v, n01v, tabv, accv, cfirst, cown, basebuf, shbase):
    cid = lax.axis_index("c")
    sid = lax.axis_index("s")
    wid = cid * NSUB + sid

    pltpu.sync_copy(src_hbm.at[wid, 0], srcv)
    pltpu.sync_copy(lrank_hbm.at[wid, 0], cmbv)
    pltpu.sync_copy(e01_hbm.at[wid, 0], e01v)
    pltpu.sync_copy(n01_hbm, n01v)
    pltpu.sync_copy(tab_hbm, tabv)

    S = sid * SL
    for k in range(NSUB):
        pltpu.sync_copy(cnt_hbm.at[k, 0, pl.ds(S, SL)],
                        cfirst.at[pl.ds(k * SL, SL)])
        pltpu.sync_copy(cnt_hbm.at[cid * NSUB + k, 0, pl.ds(S, SL)],
                        cown.at[pl.ds(k * SL, SL)])

    cidv = jnp.full((16,), 0, jnp.int32) + cid

    def pbody(i, c):
        off = i * 16
        t0 = cfirst[pl.ds(off, 16)]
        for k in range(1, NSUB):
            t0 = t0 + cfirst[pl.ds(k * SL + off, 16)]
        run = t0 * cidv
        for k in range(NSUB):
            basebuf[pl.ds(k * SL + off, 16)] = run
            run = run + cown[pl.ds(k * SL + off, 16)]
        return c

    lax.fori_loop(0, SL // 16, pbody, 0)
    pltpu.sync_copy(basebuf, shbase.at[sid, 0])
    plsc.subcore_barrier()
    for t in range(NSUB):
        pltpu.sync_copy(shbase.at[t, 0, pl.ds(sid * SL, SL)],
                        basev.at[pl.ds(t * SL, SL)])

    def zacc(i, c):
        accv[pl.ds(i * 16, 16)] = jnp.zeros((16,), jnp.float32)
        return c

    lax.fori_loop(0, (3 * NP) // 16, zacc, 0)

    def ebody(i, c):
        b = i * 16
        cmb = cmbv[pl.ds(b, 16)]
        d = lax.shift_right_logical(cmb, 13)
        lr = jnp.bitwise_and(cmb, 8191)
        sidx = srcv[pl.ds(b, 16)]
        ev = e01v[pl.ds(b, 16)]
        e0 = plsc.bitcast(jnp.bitwise_and(ev, jnp.int32(-65536)), jnp.float32)
        e1 = plsc.bitcast(lax.shift_left(ev, 16), jnp.float32)
        sv = plsc.load_gather(n01v, [sidx])
        s0 = plsc.bitcast(jnp.bitwise_and(sv, jnp.int32(-65536)), jnp.float32)
        s1 = plsc.bitcast(lax.shift_left(sv, 16), jnp.float32)
        bs = plsc.load_gather(basev, [d])
        rank = bs + lr
        keep = rank < K
        tb = jnp.minimum(rank, 2) * 12
        cf = [plsc.load_gather(tabv, [tb + kk]) for kk in range(12)]
        for o in range(3):
            co = (e0 * cf[o * 4] + e1 * cf[o * 4 + 1]
                  + s0 * cf[o * 4 + 2] + s1 * cf[o * 4 + 3])
            plsc.addupdate_scatter(accv, [o * NP + d], co, mask=keep)
        return c

    lax.fori_loop(0, CH // 16, ebody, 0)

    pltpu.sync_copy(accv, acc_hbm.at[wid, 0])


def _tail_body(nf_ref, acc_ref, cnt_ref, batch_ref, wqn_ref, wu1n_ref,
               wu1m_ref, bu1_ref, wu2_ref, bu2_ref, lng_ref, lnb_ref,
               wh1_ref, bh1_ref, wh2_ref, bh2_ref, wh3_ref, bh3_ref, o_ref):
    nfT = nf_ref[...]
    acc_in = acc_ref[...]
    planes = [jnp.sum(acc_in[:, o * NP:o * NP + N], axis=0, keepdims=True)
              for o in range(3)]
    accT = jnp.concatenate(planes, axis=0)
    acc3T = accT + _dgt(wqn_ref[...], nfT)
    msgT = jnp.cos(acc3T)
    hT = _dgt(wu1n_ref[...], nfT) + _dgt(wu1m_ref[...], msgT) + bu1_ref[...]
    hT = _leaky(hT)
    updT = _dgt(wu2_ref[...], hT) + bu2_ref[...]
    cntf = cnt_ref[...][:, :N].astype(jnp.float32)
    indeg = jnp.sum(cntf, axis=0, keepdims=True)
    unT = updT * (indeg > 0.5).astype(jnp.float32)
    m = jnp.mean(unT, axis=0, keepdims=True)
    v = jnp.mean((unT - m) ** 2, axis=0, keepdims=True)
    nf2T = ((unT - m) * lax.rsqrt(v + 1e-5) * lng_ref[...] + lnb_ref[...]
            + nfT)
    onehot = (batch_ref[...]
              == lax.broadcasted_iota(jnp.int32, (N, NG), 1)).astype(jnp.float32)
    gT = lax.dot_general(nf2T, onehot, (((1,), (0,)), ((), ())),
                         preferred_element_type=jnp.float32,
                         precision=lax.Precision.HIGHEST)
    h1T = _dgt(wh1_ref[...], gT) + bh1_ref[...]
    m1 = jnp.mean(h1T, axis=1, keepdims=True)
    v1 = jnp.mean((h1T - m1) ** 2, axis=1, keepdims=True)
    h1T = _leaky((h1T - m1) * lax.rsqrt(v1 + 1e-5))
    h2T = _dgt(wh2_ref[...], h1T) + bh2_ref[...]
    m2 = jnp.mean(h2T, axis=1, keepdims=True)
    v2 = jnp.mean((h2T - m2) ** 2, axis=1, keepdims=True)
    h2T = _leaky((h2T - m2) * lax.rsqrt(v2 + 1e-5))
    o_ref[...] = _dgt(h2T, wh3_ref[...]) + bh3_ref[...]


def kernel(node_feat, edge_attr, params, edge_index, batch):
    p = params
    f32 = jnp.float32

    nfT = pl.pallas_call(
        _node_mlp_body,
        out_shape=jax.ShapeDtypeStruct((2, N), f32),
    )(node_feat, p['Wn1'], p['bn1'].reshape(1, H), p['Wn2'],
      p['bn2'].reshape(2, 1))

    eye8 = jnp.eye(8, dtype=f32)
    ef16 = pl.pallas_call(
        _edge_mlp_body,
        grid=(2, _NB8),
        in_specs=[
            pl.BlockSpec((_EB8, H), lambda i, j: (j, 0)),
            pl.BlockSpec((H, 8 * H), lambda i, j: (0, 0)),
            pl.BlockSpec((1, 8 * H), lambda i, j: (0, 0)),
            pl.BlockSpec((8 * H, 16), lambda i, j: (0, 0)),
            pl.BlockSpec((1, 16), lambda i, j: (0, 0)),
        ],
        out_specs=pl.BlockSpec((_EB8, 16), lambda i, j: (j, 0)),
        out_shape=jax.ShapeDtypeStruct((E // 8, 16), f32),
        scratch_shapes=[pltpu.VMEM((4, 8 * H), f32)],
    )(edge_attr.reshape(E // 8, 8 * 16), jnp.kron(eye8, p['We1']),
      jnp.tile(p['be1'], 8).reshape(1, 8 * H), jnp.kron(eye8, p['We2']),
      jnp.tile(p['be2'], 8).reshape(1, 16))
    ef2 = ef16.reshape(E, 2)

    src = edge_index[0]
    dst = edge_index[1]
    pad = CH - CHR
    _rb = lambda x: x.astype(jnp.bfloat16).astype(f32)
    dstp = jnp.pad(dst.reshape(NW, CHR), ((0, 0), (0, pad)),
                   constant_values=N).reshape(NW, 1, CH)
    srcp = jnp.pad(src.reshape(NW, CHR), ((0, 0), (0, pad))).reshape(NW, 1, CH)
    def _packpair(a, b):
        au = lax.bitcast_convert_type(_rb(a), jnp.int32)
        bu = lax.bitcast_convert_type(_rb(b), jnp.int32)
        return jnp.bitwise_and(au, jnp.int32(-65536)) | lax.shift_right_logical(
            bu, 16)

    e01p = _packpair(jnp.pad(ef2[:, 0].reshape(NW, CHR), ((0, 0), (0, pad))),
                     jnp.pad(ef2[:, 1].reshape(NW, CHR), ((0, 0), (0, pad)))
                     ).reshape(NW, 1, CH)
    n01 = _packpair(nfT[0], nfT[1])
    Wq = p['Wq']
    tA = Wq[0:6].reshape(3, 2, 3)
    tB = Wq[8:14].reshape(3, 2, 3)
    tab = _rb(jnp.pad(jnp.stack([tA[:, 0, :], tA[:, 1, :], tB[:, 0, :],
                                 tB[:, 1, :]], axis=2).reshape(36), (0, 12)))

    cnt_all, lrank_all = _sc_hist(dstp)
    acc_part = _sc_accum(cnt_all, lrank_all, srcp, e01p, n01, tab)

    out = pl.pallas_call(
        _tail_body,
        out_shape=jax.ShapeDtypeStruct((NG, NC), f32),
    )(nfT, acc_part.reshape(NW, 3 * NP), cnt_all.reshape(NW, NP),
      batch.reshape(N, 1), Wq[6:8], p['Wu1'][:2],
      p['Wu1'][2:], p['bu1'].reshape(H, 1), p['Wu2'], p['bu2'].reshape(2, 1),
      p['ln_g'].reshape(2, 1), p['ln_b'].reshape(2, 1), p['Wh1'],
      p['bh1'].reshape(H, 1), p['Wh2'], p['bh2'].reshape(H, 1), p['Wh3'],
      p['bh3'].reshape(1, NC))
    return out


# speedup vs baseline: 1.2827x; 1.2827x over previous
import functools

import jax
import jax.numpy as jnp
import numpy as np
from jax import lax
from jax.experimental import pallas as pl
from jax.experimental.pallas import tpu as pltpu
from jax.experimental.pallas import tpu_sc as plsc

N = 10000
E = 160000
K = 3
NG = 64
NC = 2
H = 128

NCORE = 2
NSUB = 16
NW = NCORE * NSUB
CHR = E // NW
CH = 5008
NP = 10240
EPAD = NW * CH
LEAK = 0.01

_mesh = plsc.VectorSubcoreMesh(core_axis_name="c", subcore_axis_name="s")
_sc_params = pltpu.CompilerParams(needs_layout_passes=False)


def _leaky(x):
    return jnp.where(x >= 0, x, LEAK * x)


def _dgt(a, b):
    return lax.dot_general(a, b, (((0,), (0,)), ((), ())),
                           preferred_element_type=jnp.float32)


def _node_mlp_body(x_ref, w1_ref, b1_ref, w2_ref, b2_ref, o_ref):
    x = x_ref[...]
    h = jnp.dot(x, w1_ref[...], preferred_element_type=jnp.float32) + b1_ref[...]
    m = jnp.mean(h, axis=0, keepdims=True)
    v = jnp.mean((h - m) ** 2, axis=0, keepdims=True)
    hn = _leaky((h - m) * lax.rsqrt(v + 1e-5))
    o = lax.dot_general(w2_ref[...], hn, (((0,), (1,)), ((), ())),
                        preferred_element_type=jnp.float32) + b2_ref[...]
    o_ref[...] = jnp.tanh(o) * np.pi


_EB8 = 2000
_NB8 = (E // 8) // _EB8


def _edge_mlp_body(x_ref, w1_ref, b1_ref, w2_ref, b2_ref, o_ref, stat_ref):
    i = pl.program_id(0)
    j = pl.program_id(1)

    @pl.when(jnp.logical_and(i == 0, j == 0))
    def _():
        stat_ref[...] = jnp.zeros_like(stat_ref)

    h = (jnp.dot(x_ref[...], w1_ref[...], preferred_element_type=jnp.float32)
         + b1_ref[...])

    @pl.when(i == 0)
    def _():
        stat_ref[0:1] += jnp.sum(h, axis=0, keepdims=True)
        stat_ref[1:2] += jnp.sum(h * h, axis=0, keepdims=True)
        o_ref[...] = jnp.zeros_like(o_ref)

    @pl.when(jnp.logical_and(i == 0, j == _NB8 - 1))
    def _():
        sm = jnp.sum(stat_ref[0:1].reshape(8, H), axis=0, keepdims=True) / E
        ss = jnp.sum(stat_ref[1:2].reshape(8, H), axis=0, keepdims=True) / E
        var = ss - sm * sm
        rs = lax.rsqrt(var + 1e-5)
        stat_ref[2:3] = jnp.concatenate([sm] * 8, axis=1)
        stat_ref[3:4] = jnp.concatenate([rs] * 8, axis=1)

    @pl.when(i == 1)
    def _():
        hn = _leaky((h - stat_ref[2:3]) * stat_ref[3:4])
        o = (jnp.dot(hn, w2_ref[...], preferred_element_type=jnp.float32)
             + b2_ref[...])
        o = jnp.tanh(o) * np.pi
        a = lax.bitcast_convert_type(
            o[:, 0:8].astype(jnp.bfloat16).astype(jnp.float32), jnp.int32)
        b = lax.bitcast_convert_type(
            o[:, 8:16].astype(jnp.bfloat16).astype(jnp.float32), jnp.int32)
        o_ref[...] = (jnp.bitwise_and(a, jnp.int32(-65536))
                      | lax.shift_right_logical(b, 16))


@functools.partial(
    pl.kernel,
    out_type=(
        jax.ShapeDtypeStruct((NW, 1, NP), jnp.int32),
        jax.ShapeDtypeStruct((EPAD,), jnp.int32),
    ),
    mesh=_mesh,
    scratch_types=[
        pltpu.VMEM((CH,), jnp.int32),
        pltpu.VMEM((NP,), jnp.int32),
        pltpu.VMEM((CH,), jnp.int32),
    ],
    compiler_params=_sc_params,
)
def _sc_hist(dst_hbm, cnt_hbm, lrank_hbm, dstv, cntv, lrankv):
    cid = lax.axis_index("c")
    sid = lax.axis_index("s")
    wid = cid * NSUB + sid
    pltpu.sync_copy(dst_hbm.at[pl.ds(wid * CH, CH)], dstv)

    def zero_body(i, c):
        cntv[pl.ds(i * 16, 16)] = jnp.zeros((16,), jnp.int32)
        return c

    lax.fori_loop(0, NP // 16, zero_body, 0)

    def body(i, c):
        base = i * 16
        d = dstv[pl.ds(base, 16)]
        r0 = plsc.load_gather(cntv, [d])
        cntd, _ = plsc.scan_count(d)
        lrankv[pl.ds(base, 16)] = d * 8192 + (r0 + cntd - 1)
        plsc.store_scatter(cntv, [d], r0 + cntd)
        return c

    lax.fori_loop(0, CH // 16, body, 0)
    pltpu.sync_copy(cntv, cnt_hbm.at[wid, 0])
    pltpu.sync_copy(lrankv, lrank_hbm.at[pl.ds(wid * CH, CH)])


SL = NP // NSUB


@functools.partial(
    pl.kernel,
    out_type=jax.ShapeDtypeStruct((NW, 1, 3 * NP), jnp.float32),
    mesh=_mesh,
    scratch_types=[
        pltpu.VMEM((NP,), jnp.int32),
        pltpu.VMEM((CH,), jnp.int32),
        pltpu.VMEM((CH,), jnp.int32),
        pltpu.VMEM((CH,), jnp.int32),
        pltpu.VMEM((N,), jnp.int32),
        pltpu.VMEM((48,), jnp.float32),
        pltpu.VMEM((3 * NP,), jnp.float32),
        pltpu.VMEM((NSUB * SL,), jnp.int32),
        pltpu.VMEM((NSUB * SL,), jnp.int32),
        pltpu.VMEM((NSUB * SL,), jnp.int32),
        pltpu.VMEM_SHARED((NSUB, 1, NSUB * SL), jnp.int32),
    ],
    compiler_params=_sc_params,
)
def _sc_accum(cnt_hbm, lrank_hbm, src_hbm, e01_hbm, n01_hbm,
              tab_hbm, acc_hbm, basev, srcv, cmbv,
              e01v, n01v, tabv, accv, cfirst, cown, basebuf, shbase):
    cid = lax.axis_index("c")
    sid = lax.axis_index("s")
    wid = cid * NSUB + sid

    pltpu.sync_copy(src_hbm.at[pl.ds(wid * CH, CH)], srcv)
    pltpu.sync_copy(lrank_hbm.at[pl.ds(wid * CH, CH)], cmbv)
    pltpu.sync_copy(e01_hbm.at[pl.ds(wid * CH, CH)], e01v)
    pltpu.sync_copy(n01_hbm, n01v)
    pltpu.sync_copy(tab_hbm, tabv)

    S = sid * SL
    for k in range(NSUB):
        pltpu.sync_copy(cnt_hbm.at[k, 0, pl.ds(S, SL)],
                        cfirst.at[pl.ds(k * SL, SL)])
        pltpu.sync_copy(cnt_hbm.at[cid * NSUB + k, 0, pl.ds(S, SL)],
                        cown.at[pl.ds(k * SL, SL)])

    cidv = jnp.full((16,), 0, jnp.int32) + cid

    def pbody(i, c):
        off = i * 16
        t0 = cfirst[pl.ds(off, 16)]
        for k in range(1, NSUB):
            t0 = t0 + cfirst[pl.ds(k * SL + off, 16)]
        run = t0 * cidv
        for k in range(NSUB):
            basebuf[pl.ds(k * SL + off, 16)] = run
            run = run + cown[pl.ds(k * SL + off, 16)]
        return c

    lax.fori_loop(0, SL // 16, pbody, 0)
    pltpu.sync_copy(basebuf, shbase.at[sid, 0])
    plsc.subcore_barrier()
    for t in range(NSUB):
        pltpu.sync_copy(shbase.at[t, 0, pl.ds(sid * SL, SL)],
                        basev.at[pl.ds(t * SL, SL)])

    def zacc(i, c):
        accv[pl.ds(i * 16, 16)] = jnp.zeros((16,), jnp.float32)
        return c

    lax.fori_loop(0, (3 * NP) // 16, zacc, 0)

    def ebody(i, c):
        b = i * 16
        cmb = cmbv[pl.ds(b, 16)]
        d = lax.shift_right_logical(cmb, 13)
        lr = jnp.bitwise_and(cmb, 8191)
        sidx = srcv[pl.ds(b, 16)]
        ev = e01v[pl.ds(b, 16)]
        e0 = plsc.bitcast(jnp.bitwise_and(ev, jnp.int32(-65536)), jnp.float32)
        e1 = plsc.bitcast(lax.shift_left(ev, 16), jnp.float32)
        sv = plsc.load_gather(n01v, [sidx])
        s0 = plsc.bitcast(jnp.bitwise_and(sv, jnp.int32(-65536)), jnp.float32)
        s1 = plsc.bitcast(lax.shift_left(sv, 16), jnp.float32)
        bs = plsc.load_gather(basev, [d])
        rank = bs + lr
        keep = rank < K
        tb = jnp.minimum(rank, 2) * 12
        cf = [plsc.load_gather(tabv, [tb + kk]) for kk in range(12)]
        for o in range(3):
            co = (e0 * cf[o * 4] + e1 * cf[o * 4 + 1]
                  + s0 * cf[o * 4 + 2] + s1 * cf[o * 4 + 3])
            plsc.addupdate_scatter(accv, [o * NP + d], co, mask=keep)
        return c

    lax.fori_loop(0, CH // 16, ebody, 0)

    pltpu.sync_copy(accv, acc_hbm.at[wid, 0])


def _tail_body(nf_ref, acc_ref, cnt_ref, batch_ref, wqn_ref, wu1n_ref,
               wu1m_ref, bu1_ref, wu2_ref, bu2_ref, lng_ref, lnb_ref,
               wh1_ref, bh1_ref, wh2_ref, bh2_ref, wh3_ref, bh3_ref, o_ref):
    nfT = nf_ref[...]
    acc_in = acc_ref[...]
    planes = [jnp.sum(acc_in[:, o * NP:o * NP + N], axis=0, keepdims=True)
              for o in range(3)]
    accT = jnp.concatenate(planes, axis=0)
    acc3T = accT + _dgt(wqn_ref[...], nfT)
    msgT = jnp.cos(acc3T)
    hT = _dgt(wu1n_ref[...], nfT) + _dgt(wu1m_ref[...], msgT) + bu1_ref[...]
    hT = _leaky(hT)
    updT = _dgt(wu2_ref[...], hT) + bu2_ref[...]
    cntf = cnt_ref[...][:, :N].astype(jnp.float32)
    indeg = jnp.sum(cntf, axis=0, keepdims=True)
    unT = updT * (indeg > 0.5).astype(jnp.float32)
    m = jnp.mean(unT, axis=0, keepdims=True)
    v = jnp.mean((unT - m) ** 2, axis=0, keepdims=True)
    nf2T = ((unT - m) * lax.rsqrt(v + 1e-5) * lng_ref[...] + lnb_ref[...]
            + nfT)
    onehot = (batch_ref[...]
              == lax.broadcasted_iota(jnp.int32, (N, NG), 1)).astype(jnp.float32)
    gT = lax.dot_general(nf2T, onehot, (((1,), (0,)), ((), ())),
                         preferred_element_type=jnp.float32,
                         precision=lax.Precision.HIGHEST)
    h1T = _dgt(wh1_ref[...], gT) + bh1_ref[...]
    m1 = jnp.mean(h1T, axis=1, keepdims=True)
    v1 = jnp.mean((h1T - m1) ** 2, axis=1, keepdims=True)
    h1T = _leaky((h1T - m1) * lax.rsqrt(v1 + 1e-5))
    h2T = _dgt(wh2_ref[...], h1T) + bh2_ref[...]
    m2 = jnp.mean(h2T, axis=1, keepdims=True)
    v2 = jnp.mean((h2T - m2) ** 2, axis=1, keepdims=True)
    h2T = _leaky((h2T - m2) * lax.rsqrt(v2 + 1e-5))
    o_ref[...] = _dgt(h2T, wh3_ref[...]) + bh3_ref[...]


def kernel(node_feat, edge_attr, params, edge_index, batch):
    p = params
    f32 = jnp.float32

    nfT = pl.pallas_call(
        _node_mlp_body,
        out_shape=jax.ShapeDtypeStruct((2, N), f32),
    )(node_feat, p['Wn1'], p['bn1'].reshape(1, H), p['Wn2'],
      p['bn2'].reshape(2, 1))

    eye8 = jnp.eye(8, dtype=f32)
    w2g = jnp.concatenate([jnp.kron(eye8, p['We2'][:, 0:1]),
                           jnp.kron(eye8, p['We2'][:, 1:2])], axis=1)
    b2g = jnp.concatenate([jnp.full((8,), p['be2'][0], f32),
                           jnp.full((8,), p['be2'][1], f32)]).reshape(1, 16)
    e01_8 = pl.pallas_call(
        _edge_mlp_body,
        grid=(2, _NB8),
        in_specs=[
            pl.BlockSpec((_EB8, H), lambda i, j: (j, 0)),
            pl.BlockSpec((H, 8 * H), lambda i, j: (0, 0)),
            pl.BlockSpec((1, 8 * H), lambda i, j: (0, 0)),
            pl.BlockSpec((8 * H, 16), lambda i, j: (0, 0)),
            pl.BlockSpec((1, 16), lambda i, j: (0, 0)),
        ],
        out_specs=pl.BlockSpec((_EB8, 8), lambda i, j: (j, 0)),
        out_shape=jax.ShapeDtypeStruct((E // 8, 8), jnp.int32),
        scratch_shapes=[pltpu.VMEM((4, 8 * H), f32)],
    )(edge_attr.reshape(E // 8, 8 * 16), jnp.kron(eye8, p['We1']),
      jnp.tile(p['be1'], 8).reshape(1, 8 * H), w2g, b2g)

    src = edge_index[0]
    dst = edge_index[1]
    _rb = lambda x: x.astype(jnp.bfloat16).astype(f32)
    dstp = jnp.pad(dst, (0, EPAD - E), constant_values=N)
    srcp = jnp.pad(src, (0, EPAD - E))
    e01p = jnp.pad(e01_8.reshape(E), (0, EPAD - E))

    def _packpair(a, b):
        au = lax.bitcast_convert_type(_rb(a), jnp.int32)
        bu = lax.bitcast_convert_type(_rb(b), jnp.int32)
        return jnp.bitwise_and(au, jnp.int32(-65536)) | lax.shift_right_logical(
            bu, 16)

    n01 = _packpair(nfT[0], nfT[1])
    Wq = p['Wq']
    tA = Wq[0:6].reshape(3, 2, 3)
    tB = Wq[8:14].reshape(3, 2, 3)
    tab = _rb(jnp.pad(jnp.stack([tA[:, 0, :], tA[:, 1, :], tB[:, 0, :],
                                 tB[:, 1, :]], axis=2).reshape(36), (0, 12)))

    cnt_all, lrank_all = _sc_hist(dstp)
    acc_part = _sc_accum(cnt_all, lrank_all, srcp, e01p, n01, tab)

    out = pl.pallas_call(
        _tail_body,
        out_shape=jax.ShapeDtypeStruct((NG, NC), f32),
    )(nfT, acc_part.reshape(NW, 3 * NP), cnt_all.reshape(NW, NP),
      batch.reshape(N, 1), Wq[6:8], p['Wu1'][:2],
      p['Wu1'][2:], p['bu1'].reshape(H, 1), p['Wu2'], p['bu2'].reshape(2, 1),
      p['ln_g'].reshape(2, 1), p['ln_b'].reshape(2, 1), p['Wh1'],
      p['bh1'].reshape(H, 1), p['Wh2'], p['bh2'].reshape(H, 1), p['Wh3'],
      p['bh3'].reshape(1, NC))
    return out


# speedup vs baseline: 1.3372x; 1.0425x over previous
import functools

import jax
import jax.numpy as jnp
import numpy as np
from jax import lax
from jax.experimental import pallas as pl
from jax.experimental.pallas import tpu as pltpu
from jax.experimental.pallas import tpu_sc as plsc

N = 10000
E = 160000
K = 3
NG = 64
NC = 2
H = 128

NCORE = 2
NSUB = 16
NW = NCORE * NSUB
CHR = E // NW
CH = 5008
NP = 10240
EPAD = NW * CH
LEAK = 0.01

_mesh = plsc.VectorSubcoreMesh(core_axis_name="c", subcore_axis_name="s")
_sc_params = pltpu.CompilerParams(needs_layout_passes=False)


def _leaky(x):
    return jnp.where(x >= 0, x, LEAK * x)


def _dgt(a, b):
    return lax.dot_general(a, b, (((0,), (0,)), ((), ())),
                           preferred_element_type=jnp.float32)


def _node_mlp_body(x_ref, w1_ref, b1_ref, w2_ref, b2_ref, o_ref):
    x = x_ref[...]
    h = jnp.dot(x, w1_ref[...], preferred_element_type=jnp.float32) + b1_ref[...]
    m = jnp.mean(h, axis=0, keepdims=True)
    v = jnp.mean((h - m) ** 2, axis=0, keepdims=True)
    hn = _leaky((h - m) * lax.rsqrt(v + 1e-5))
    o = lax.dot_general(w2_ref[...], hn, (((0,), (1,)), ((), ())),
                        preferred_element_type=jnp.float32) + b2_ref[...]
    o_ref[...] = jnp.tanh(o) * np.pi


_EB8 = 2000
_NB8 = (E // 8) // _EB8


def _edge_mlp_body(x_ref, w1_ref, b1_ref, w2_ref, b2_ref, o_ref, stat_ref):
    i = pl.program_id(0)
    j = pl.program_id(1)

    @pl.when(jnp.logical_and(i == 0, j == 0))
    def _():
        stat_ref[...] = jnp.zeros_like(stat_ref)

    h = (jnp.dot(x_ref[...], w1_ref[...], preferred_element_type=jnp.float32)
         + b1_ref[...])

    @pl.when(i == 0)
    def _():
        stat_ref[0:1] += jnp.sum(h, axis=0, keepdims=True)
        stat_ref[1:2] += jnp.sum(h * h, axis=0, keepdims=True)
        o_ref[...] = jnp.zeros_like(o_ref)

    @pl.when(jnp.logical_and(i == 0, j == _NB8 - 1))
    def _():
        sm = jnp.sum(stat_ref[0:1].reshape(8, H), axis=0, keepdims=True) / E
        ss = jnp.sum(stat_ref[1:2].reshape(8, H), axis=0, keepdims=True) / E
        var = ss - sm * sm
        rs = lax.rsqrt(var + 1e-5)
        stat_ref[2:3] = jnp.concatenate([sm] * 8, axis=1)
        stat_ref[3:4] = jnp.concatenate([rs] * 8, axis=1)

    @pl.when(i == 1)
    def _():
        hn = _leaky((h - stat_ref[2:3]) * stat_ref[3:4])
        o = (jnp.dot(hn, w2_ref[...], preferred_element_type=jnp.float32)
             + b2_ref[...])
        o = jnp.tanh(o) * np.pi
        a = lax.bitcast_convert_type(
            o[:, 0:8].astype(jnp.bfloat16).astype(jnp.float32), jnp.int32)
        b = lax.bitcast_convert_type(
            o[:, 8:16].astype(jnp.bfloat16).astype(jnp.float32), jnp.int32)
        o_ref[...] = (jnp.bitwise_and(a, jnp.int32(-65536))
                      | lax.shift_right_logical(b, 16))


@functools.partial(
    pl.kernel,
    out_type=(
        jax.ShapeDtypeStruct((NW, 1, NP), jnp.int32),
        jax.ShapeDtypeStruct((EPAD,), jnp.int32),
    ),
    mesh=_mesh,
    scratch_types=[
        pltpu.VMEM((CH,), jnp.int32),
        pltpu.VMEM((NP,), jnp.int32),
        pltpu.VMEM((CH,), jnp.int32),
    ],
    compiler_params=_sc_params,
)
def _sc_hist(dst_hbm, cnt_hbm, lrank_hbm, dstv, cntv, lrankv):
    cid = lax.axis_index("c")
    sid = lax.axis_index("s")
    wid = cid * NSUB + sid
    pltpu.sync_copy(dst_hbm.at[pl.ds(wid * CH, CH)], dstv)

    def zero_body(i, c):
        cntv[pl.ds(i * 16, 16)] = jnp.zeros((16,), jnp.int32)
        return c

    lax.fori_loop(0, NP // 16, zero_body, 0)

    def body(i, c):
        base = i * 16
        d = dstv[pl.ds(base, 16)]
        r0 = plsc.load_gather(cntv, [d])
        cntd, _ = plsc.scan_count(d)
        lrankv[pl.ds(base, 16)] = d * 8192 + (r0 + cntd - 1)
        plsc.store_scatter(cntv, [d], r0 + cntd)
        return c

    lax.fori_loop(0, CH // 16, body, 0)
    pltpu.sync_copy(cntv, cnt_hbm.at[wid, 0])
    pltpu.sync_copy(lrankv, lrank_hbm.at[pl.ds(wid * CH, CH)])


SL = NP // NSUB


@functools.partial(
    pl.kernel,
    out_type=jax.ShapeDtypeStruct((NW, 1, 3 * NP), jnp.float32),
    mesh=_mesh,
    scratch_types=[
        pltpu.VMEM((NP,), jnp.int32),
        pltpu.VMEM((CH,), jnp.int32),
        pltpu.VMEM((CH,), jnp.int32),
        pltpu.VMEM((CH,), jnp.int32),
        pltpu.VMEM((N,), jnp.int32),
        pltpu.VMEM((48,), jnp.float32),
        pltpu.VMEM((3 * NP,), jnp.float32),
        pltpu.VMEM((NSUB * SL,), jnp.int32),
        pltpu.VMEM((NSUB * SL,), jnp.int32),
        pltpu.VMEM((NSUB * SL,), jnp.int32),
        pltpu.VMEM_SHARED((NSUB, 1, NSUB * SL), jnp.int32),
    ],
    compiler_params=_sc_params,
)
def _sc_accum(cnt_hbm, lrank_hbm, src_hbm, e01_hbm, n01_hbm,
              tab_hbm, acc_hbm, basev, srcv, cmbv,
              e01v, n01v, tabv, accv, cfirst, cown, basebuf, shbase):
    cid = lax.axis_index("c")
    sid = lax.axis_index("s")
    wid = cid * NSUB + sid

    pltpu.sync_copy(src_hbm.at[pl.ds(wid * CH, CH)], srcv)
    pltpu.sync_copy(lrank_hbm.at[pl.ds(wid * CH, CH)], cmbv)
    pltpu.sync_copy(e01_hbm.at[pl.ds(wid * CH, CH)], e01v)
    pltpu.sync_copy(n01_hbm, n01v)
    pltpu.sync_copy(tab_hbm, tabv)

    S = sid * SL
    for k in range(NSUB):
        pltpu.sync_copy(cnt_hbm.at[k, 0, pl.ds(S, SL)],
                        cfirst.at[pl.ds(k * SL, SL)])
        pltpu.sync_copy(cnt_hbm.at[cid * NSUB + k, 0, pl.ds(S, SL)],
                        cown.at[pl.ds(k * SL, SL)])

    cidv = jnp.full((16,), 0, jnp.int32) + cid

    def pbody(i, c):
        off = i * 16
        t0 = cfirst[pl.ds(off, 16)]
        for k in range(1, NSUB):
            t0 = t0 + cfirst[pl.ds(k * SL + off, 16)]
        run = t0 * cidv
        for k in range(NSUB):
            basebuf[pl.ds(k * SL + off, 16)] = run
            run = run + cown[pl.ds(k * SL + off, 16)]
        return c

    lax.fori_loop(0, SL // 16, pbody, 0)
    pltpu.sync_copy(basebuf, shbase.at[sid, 0])
    plsc.subcore_barrier()
    for t in range(NSUB):
        pltpu.sync_copy(shbase.at[t, 0, pl.ds(sid * SL, SL)],
                        basev.at[pl.ds(t * SL, SL)])

    def zacc(i, c):
        accv[pl.ds(i * 16, 16)] = jnp.zeros((16,), jnp.float32)
        return c

    lax.fori_loop(0, (3 * NP) // 16, zacc, 0)

    @functools.partial(plsc.parallel_loop, 0, CH // 16, unroll=2)
    def ebody(i):
        b = i * 16
        cmb = cmbv[pl.ds(b, 16)]
        d = lax.shift_right_logical(cmb, 13)
        lr = jnp.bitwise_and(cmb, 8191)
        sidx = srcv[pl.ds(b, 16)]
        ev = e01v[pl.ds(b, 16)]
        e0 = plsc.bitcast(jnp.bitwise_and(ev, jnp.int32(-65536)), jnp.float32)
        e1 = plsc.bitcast(lax.shift_left(ev, 16), jnp.float32)
        sv = plsc.load_gather(n01v, [sidx])
        s0 = plsc.bitcast(jnp.bitwise_and(sv, jnp.int32(-65536)), jnp.float32)
        s1 = plsc.bitcast(lax.shift_left(sv, 16), jnp.float32)
        bs = plsc.load_gather(basev, [d])
        rank = bs + lr
        keep = rank < K
        tb = jnp.minimum(rank, 2) * 12
        cf = [plsc.load_gather(tabv, [tb + kk]) for kk in range(12)]
        for o in range(3):
            co = (e0 * cf[o * 4] + e1 * cf[o * 4 + 1]
                  + s0 * cf[o * 4 + 2] + s1 * cf[o * 4 + 3])
            plsc.addupdate_scatter(accv, [o * NP + d], co, mask=keep)

    pltpu.sync_copy(accv, acc_hbm.at[wid, 0])


def _tail_body(nf_ref, acc_ref, cnt_ref, batch_ref, wqn_ref, wu1n_ref,
               wu1m_ref, bu1_ref, wu2_ref, bu2_ref, lng_ref, lnb_ref,
               wh1_ref, bh1_ref, wh2_ref, bh2_ref, wh3_ref, bh3_ref, o_ref):
    nfT = nf_ref[...]
    acc_in = acc_ref[...]
    planes = [jnp.sum(acc_in[:, o * NP:o * NP + N], axis=0, keepdims=True)
              for o in range(3)]
    accT = jnp.concatenate(planes, axis=0)
    acc3T = accT + _dgt(wqn_ref[...], nfT)
    msgT = jnp.cos(acc3T)
    hT = _dgt(wu1n_ref[...], nfT) + _dgt(wu1m_ref[...], msgT) + bu1_ref[...]
    hT = _leaky(hT)
    updT = _dgt(wu2_ref[...], hT) + bu2_ref[...]
    cntf = cnt_ref[...][:, :N].astype(jnp.float32)
    indeg = jnp.sum(cntf, axis=0, keepdims=True)
    unT = updT * (indeg > 0.5).astype(jnp.float32)
    m = jnp.mean(unT, axis=0, keepdims=True)
    v = jnp.mean((unT - m) ** 2, axis=0, keepdims=True)
    nf2T = ((unT - m) * lax.rsqrt(v + 1e-5) * lng_ref[...] + lnb_ref[...]
            + nfT)
    onehot = (batch_ref[...]
              == lax.broadcasted_iota(jnp.int32, (N, NG), 1)).astype(jnp.float32)
    gT = lax.dot_general(nf2T, onehot, (((1,), (0,)), ((), ())),
                         preferred_element_type=jnp.float32,
                         precision=lax.Precision.HIGHEST)
    h1T = _dgt(wh1_ref[...], gT) + bh1_ref[...]
    m1 = jnp.mean(h1T, axis=1, keepdims=True)
    v1 = jnp.mean((h1T - m1) ** 2, axis=1, keepdims=True)
    h1T = _leaky((h1T - m1) * lax.rsqrt(v1 + 1e-5))
    h2T = _dgt(wh2_ref[...], h1T) + bh2_ref[...]
    m2 = jnp.mean(h2T, axis=1, keepdims=True)
    v2 = jnp.mean((h2T - m2) ** 2, axis=1, keepdims=True)
    h2T = _leaky((h2T - m2) * lax.rsqrt(v2 + 1e-5))
    o_ref[...] = _dgt(h2T, wh3_ref[...]) + bh3_ref[...]


def kernel(node_feat, edge_attr, params, edge_index, batch):
    p = params
    f32 = jnp.float32

    nfT = pl.pallas_call(
        _node_mlp_body,
        out_shape=jax.ShapeDtypeStruct((2, N), f32),
    )(node_feat, p['Wn1'], p['bn1'].reshape(1, H), p['Wn2'],
      p['bn2'].reshape(2, 1))

    eye8 = jnp.eye(8, dtype=f32)
    w2g = jnp.concatenate([jnp.kron(eye8, p['We2'][:, 0:1]),
                           jnp.kron(eye8, p['We2'][:, 1:2])], axis=1)
    b2g = jnp.concatenate([jnp.full((8,), p['be2'][0], f32),
                           jnp.full((8,), p['be2'][1], f32)]).reshape(1, 16)
    e01_8 = pl.pallas_call(
        _edge_mlp_body,
        grid=(2, _NB8),
        in_specs=[
            pl.BlockSpec((_EB8, H), lambda i, j: (j, 0)),
            pl.BlockSpec((H, 8 * H), lambda i, j: (0, 0)),
            pl.BlockSpec((1, 8 * H), lambda i, j: (0, 0)),
            pl.BlockSpec((8 * H, 16), lambda i, j: (0, 0)),
            pl.BlockSpec((1, 16), lambda i, j: (0, 0)),
        ],
        out_specs=pl.BlockSpec((_EB8, 8), lambda i, j: (j, 0)),
        out_shape=jax.ShapeDtypeStruct((E // 8, 8), jnp.int32),
        scratch_shapes=[pltpu.VMEM((4, 8 * H), f32)],
    )(edge_attr.reshape(E // 8, 8 * 16), jnp.kron(eye8, p['We1']),
      jnp.tile(p['be1'], 8).reshape(1, 8 * H), w2g, b2g)

    src = edge_index[0]
    dst = edge_index[1]
    _rb = lambda x: x.astype(jnp.bfloat16).astype(f32)
    dstp = jnp.pad(dst, (0, EPAD - E), constant_values=N)
    srcp = jnp.pad(src, (0, EPAD - E))
    e01p = jnp.pad(e01_8.reshape(E), (0, EPAD - E))

    def _packpair(a, b):
        au = lax.bitcast_convert_type(_rb(a), jnp.int32)
        bu = lax.bitcast_convert_type(_rb(b), jnp.int32)
        return jnp.bitwise_and(au, jnp.int32(-65536)) | lax.shift_right_logical(
            bu, 16)

    n01 = _packpair(nfT[0], nfT[1])
    Wq = p['Wq']
    tA = Wq[0:6].reshape(3, 2, 3)
    tB = Wq[8:14].reshape(3, 2, 3)
    tab = _rb(jnp.pad(jnp.stack([tA[:, 0, :], tA[:, 1, :], tB[:, 0, :],
                                 tB[:, 1, :]], axis=2).reshape(36), (0, 12)))

    cnt_all, lrank_all = _sc_hist(dstp)
    acc_part = _sc_accum(cnt_all, lrank_all, srcp, e01p, n01, tab)

    out = pl.pallas_call(
        _tail_body,
        out_shape=jax.ShapeDtypeStruct((NG, NC), f32),
    )(nfT, acc_part.reshape(NW, 3 * NP), cnt_all.reshape(NW, NP),
      batch.reshape(N, 1), Wq[6:8], p['Wu1'][:2],
      p['Wu1'][2:], p['bu1'].reshape(H, 1), p['Wu2'], p['bu2'].reshape(2, 1),
      p['ln_g'].reshape(2, 1), p['ln_b'].reshape(2, 1), p['Wh1'],
      p['bh1'].reshape(H, 1), p['Wh2'], p['bh2'].reshape(H, 1), p['Wh3'],
      p['bh3'].reshape(1, NC))
    return out
